# baseline (speedup 1.0000x reference)
import functools

import jax
import jax.numpy as jnp
from jax import lax
from jax.experimental import pallas as pl
from jax.experimental.pallas import tpu as pltpu
from jax.experimental.pallas import tpu_sc as plsc

N = 10000
NP = 10240
E = 320000
EP = 327680
D_FEAT = 128
D1 = 128
D2 = 64
B = 8
T = 16
NUM_CLASSES = 2

NC = 2
NS = 16
LANES = 16
NW = NC * NS
CH = 128
TRIPS = EP // (NW * CH)
RPT = NP // NS

_MESH = plsc.VectorSubcoreMesh(core_axis_name="c", subcore_axis_name="s")
_SC_PARAMS = pltpu.CompilerParams(needs_layout_passes=False)


NCH_REAL = E // CH
NCH_TOT = EP // CH
MROW = 3 * CH


@functools.partial(
    pl.kernel,
    out_type=(
        jax.ShapeDtypeStruct((NC, NP), jnp.float32),
        jax.ShapeDtypeStruct((NCH_TOT * MROW,), jnp.int32),
    ),
    mesh=_MESH,
    scratch_types=(
        pltpu.VMEM((2, CH), jnp.int32),
        pltpu.VMEM((2, CH), jnp.int32),
        pltpu.VMEM((2, CH * 4), jnp.float32),
        pltpu.VMEM((2, MROW), jnp.int32),
        pltpu.VMEM((2, CH), jnp.int32),
        pltpu.VMEM((2, CH), jnp.float32),
        pltpu.VMEM((RPT,), jnp.float32),
        pltpu.VMEM_SHARED((NP,), jnp.float32),
        pltpu.SemaphoreType.DMA,
        pltpu.SemaphoreType.DMA,
        pltpu.SemaphoreType.DMA,
        pltpu.SemaphoreType.DMA,
        pltpu.SemaphoreType.DMA,
        pltpu.SemaphoreType.DMA,
    ),
    compiler_params=_SC_PARAMS,
)
def _sc_degree(src_hbm, dst_hbm, ea_hbm, deg_out, meta_out,
               src_v, dst_v, ea_v, stage_v, dsti_v, ewf_v, zv, deg_sh,
               semi0, semi1, semw0, semw1, semc0, semc1):
    c = lax.axis_index("c")
    s = lax.axis_index("s")
    w = c * NS + s
    semi = (semi0, semi1)
    semw = (semw0, semw1)
    semc = (semc0, semc1)
    rt = 78 + jnp.where(w < NCH_REAL - 78 * NW, 1, 0)

    z16 = jnp.zeros((LANES,), jnp.float32)

    def zbody(i, carry):
        zv[pl.ds(i * LANES, LANES)] = z16
        return carry

    lax.fori_loop(0, RPT // LANES, zbody, 0)
    pltpu.sync_copy(zv, deg_sh.at[pl.ds(s * RPT, RPT)])
    plsc.subcore_barrier()

    lane_iota = lax.iota(jnp.int32, LANES)

    def issue_in(k, b):
        base = (w + k * NW) * CH
        pltpu.async_copy(src_hbm.at[pl.ds(base, CH)], src_v.at[b], semi[b])
        pltpu.async_copy(dst_hbm.at[pl.ds(base, CH)], dst_v.at[b], semi[b])
        pltpu.async_copy(ea_hbm.at[pl.ds(base * 4, CH * 4)], ea_v.at[b],
                         semi[b])

    def wait_in(b):
        pltpu.make_async_copy(src_hbm.at[pl.ds(0, CH)], src_v.at[b],
                              semi[b]).wait()
        pltpu.make_async_copy(dst_hbm.at[pl.ds(0, CH)], dst_v.at[b],
                              semi[b]).wait()
        pltpu.make_async_copy(ea_hbm.at[pl.ds(0, CH * 4)], ea_v.at[b],
                              semi[b]).wait()

    def meta_slot(k):
        return (w + k * NW) * MROW

    def wait_wr(b):
        pltpu.make_async_copy(stage_v.at[b], meta_out.at[pl.ds(0, MROW)],
                              semw[b]).wait()

    def wait_sc(b):
        pltpu.make_async_copy(ewf_v.at[b], deg_sh.at[dsti_v.at[b]],
                              semc[b]).wait()

    def body(k, b):
        in_range = k < rt

        @pl.when(in_range)
        def _():
            wait_in(b)

        @pl.when(jnp.logical_and(k < rt, k >= 2))
        def _():
            wait_wr(b)
            wait_sc(b)

        @pl.when(in_range)
        def _():
            for g in range(CH // LANES):
                sl = pl.ds(g * LANES, LANES)
                idx = lane_iota * 4 + (g * LANES * 4)
                ew16 = jnp.abs(plsc.load_gather(
                    ea_v, [jnp.full((LANES,), b, jnp.int32), idx]))
                stage_v[b, pl.ds(2 * CH + g * LANES, LANES)] = plsc.bitcast(
                    ew16, jnp.int32)
                ewf_v[b, sl] = ew16
                stage_v[b, sl] = src_v[b, sl]
                stage_v[b, pl.ds(CH + g * LANES, LANES)] = dst_v[b, sl]
                dsti_v[b, sl] = dst_v[b, sl]

        @pl.when(k + 2 < rt)
        def _():
            issue_in(k + 2, b)

        @pl.when(in_range)
        def _():
            pltpu.async_copy(stage_v.at[b],
                             meta_out.at[pl.ds(meta_slot(k), MROW)], semw[b])
            pltpu.async_copy(ewf_v.at[b], deg_sh.at[dsti_v.at[b]], semc[b],
                             add=True)

    issue_in(0, 0)
    issue_in(1, 1)

    def pair(j, carry):
        body(2 * j, 0)
        body(2 * j + 1, 1)
        return carry

    lax.fori_loop(0, (TRIPS + 1) // 2, pair, 0)
    wait_wr(0)
    wait_wr(1)
    wait_sc(0)
    wait_sc(1)

    def pad_chunk(p):
        for g in range(CH // LANES):
            sl = pl.ds(g * LANES, LANES)
            idx = N + ((lane_iota + g * LANES + p * CH) % (NP - N))
            stage_v[0, sl] = idx
            stage_v[0, pl.ds(CH + g * LANES, LANES)] = idx
            stage_v[0, pl.ds(2 * CH + g * LANES, LANES)] = jnp.zeros(
                (LANES,), jnp.int32)
        pltpu.sync_copy(stage_v.at[0], meta_out.at[pl.ds(p * MROW, MROW)])

    pad_chunk(NCH_REAL + w)

    @pl.when(w < NCH_TOT - NCH_REAL - NW)
    def _():
        pad_chunk(NCH_REAL + NW + w)

    plsc.subcore_barrier()
    pltpu.sync_copy(deg_sh.at[pl.ds(s * RPT, RPT)],
                    deg_out.at[c, pl.ds(s * RPT, RPT)])


@functools.partial(
    pl.kernel,
    out_type=jax.ShapeDtypeStruct((NC, NP, D1), jnp.float32),
    mesh=_MESH,
    scratch_types=(
        pltpu.VMEM((2, MROW), jnp.int32),
        pltpu.VMEM((2, CH), jnp.int32),
        pltpu.VMEM((2, CH), jnp.float32),
        pltpu.VMEM((2, CH, D1), jnp.float32),
        pltpu.VMEM((32, D1), jnp.float32),
        pltpu.VMEM((NP,), jnp.float32),
        pltpu.VMEM_SHARED((NP, D1), jnp.float32),
        pltpu.SemaphoreType.DMA,
        pltpu.SemaphoreType.DMA,
        pltpu.SemaphoreType.DMA,
        pltpu.SemaphoreType.DMA,
        pltpu.SemaphoreType.DMA,
        pltpu.SemaphoreType.DMA,
        pltpu.SemaphoreType.DMA,
    ),
    compiler_params=_SC_PARAMS,
)
def _sc_conv(g_hbm, meta_hbm, deg_hbm, out_hbm,
             meta_v, dsts_v, ews_v, rows_v, zrow_v, dinv_v,
             acc_sh, semm0, semm1, semg0, semg1, semt0, semt1, semz):
    c = lax.axis_index("c")
    s = lax.axis_index("s")
    w = c * NS + s
    semm = (semm0, semm1)
    semg = (semg0, semg1)
    semt = (semt0, semt1)

    z16 = jnp.zeros((LANES,), jnp.float32)

    def zbody(i, carry):
        for f in range(D1 // LANES):
            zrow_v[i, pl.ds(f * LANES, LANES)] = z16
        return carry

    lax.fori_loop(0, 32, zbody, 0)
    for i in range(RPT // 32):
        pltpu.async_copy(zrow_v, acc_sh.at[pl.ds(s * RPT + i * 32, 32)], semz)

    pltpu.sync_copy(deg_hbm.at[0], rows_v.at[0, pl.ds(0, NP // 128)])
    pltpu.sync_copy(deg_hbm.at[1], rows_v.at[1, pl.ds(0, NP // 128)])

    def dbody(i, carry):
        for f in range(128 // LANES):
            sl = pl.ds(f * LANES, LANES)
            dval = rows_v[0, i, sl] + rows_v[1, i, sl] + 1.0
            yi = jnp.int32(0x5F3759DF) - lax.shift_right_logical(
                plsc.bitcast(dval, jnp.int32), 1)
            y = plsc.bitcast(yi, jnp.float32)
            half = -0.5 * dval
            for _ in range(3):
                y = y * (1.5 + half * y * y)
            dinv_v[pl.ds(i * 128 + f * LANES, LANES)] = y
        return carry

    lax.fori_loop(0, NP // 128, dbody, 0)
    for i in range(RPT // 32):
        pltpu.make_async_copy(
            zrow_v, acc_sh.at[pl.ds(s * RPT + i * 32, 32)], semz).wait()
    plsc.subcore_barrier()

    def issue_meta(k, b):
        base = (w + k * NW) * MROW
        pltpu.async_copy(meta_hbm.at[pl.ds(base, MROW)], meta_v.at[b],
                         semm[b])

    def wait_meta(b):
        pltpu.make_async_copy(meta_hbm.at[pl.ds(0, MROW)], meta_v.at[b],
                              semm[b]).wait()

    def issue_gather(b):
        pltpu.async_copy(g_hbm.at[meta_v.at[b, pl.ds(0, CH)]], rows_v.at[b],
                         semg[b])

    def wait_gather(b):
        pltpu.make_async_copy(g_hbm.at[meta_v.at[b, pl.ds(0, CH)]],
                              rows_v.at[b], semg[b]).wait()

    def issue_scatter(b):
        pltpu.async_copy(rows_v.at[b], acc_sh.at[dsts_v.at[b]], semt[b],
                         add=True)

    def wait_scatter(b):
        pltpu.make_async_copy(rows_v.at[b], acc_sh.at[dsts_v.at[b]],
                              semt[b]).wait()

    def scale(b):
        def grp(gq, carry2):
            w16 = ews_v[b, pl.ds(gq * LANES, LANES)]
            for e in range(LANES):
                bc = jnp.take_along_axis(
                    w16, jnp.full((LANES,), e, jnp.int32), axis=0)
                r = gq * LANES + e
                for f in range(D1 // LANES):
                    sl = pl.ds(f * LANES, LANES)
                    rows_v[b, r, sl] = rows_v[b, r, sl] * bc
            return carry2

        lax.fori_loop(0, CH // LANES, grp, 0)

    def body(k, b, first, last):
        nb = 1 - b
        wait_gather(b)
        for f in range(CH // LANES):
            sl = pl.ds(f * LANES, LANES)
            dsts_v[b, sl] = meta_v[b, pl.ds(CH + f * LANES, LANES)]
            src16 = meta_v[b, sl]
            ew16 = plsc.bitcast(
                meta_v[b, pl.ds(2 * CH + f * LANES, LANES)], jnp.float32)
            ews_v[b, sl] = ew16 * plsc.load_gather(dinv_v, [src16])

        @pl.when(k + 1 < TRIPS)
        def _():
            wait_meta(nb)

        if first:
            pass
        else:
            wait_scatter(nb)

        @pl.when(k + 1 < TRIPS)
        def _():
            issue_gather(nb)

        @pl.when(k + 2 < TRIPS)
        def _():
            issue_meta(k + 2, b)

        scale(b)
        issue_scatter(b)

    issue_meta(0, 0)
    issue_meta(1, 1)
    wait_meta(0)
    issue_gather(0)

    def pair(j, carry):
        body(2 * j, 0, first=False, last=False)
        body(2 * j + 1, 1, first=False, last=False)
        return carry

    body(0, 0, first=True, last=False)
    body(1, 1, first=False, last=False)
    lax.fori_loop(1, TRIPS // 2, pair, 0)
    wait_scatter(1)
    plsc.subcore_barrier()
    pltpu.sync_copy(acc_sh.at[pl.ds(s * RPT, RPT)],
                    out_hbm.at[c, pl.ds(s * RPT, RPT)])


def _tc_matmul_pad(x_ref, w_ref, o_ref):
    o_ref[pl.ds(0, N), :] = jnp.dot(x_ref[...], w_ref[...],
                                    preferred_element_type=jnp.float32)
    o_ref[pl.ds(N, NP - N), :] = jnp.zeros((NP - N, D1), jnp.float32)


def _dinv(dp0, dp1):
    return lax.rsqrt(dp0 + dp1 + 1.0)


def _tc_combine_matmul(acc_ref, g_ref, dp0_ref, dp1_ref, b_ref, w_ref,
                       g2_ref):
    dinv = _dinv(dp0_ref[...], dp1_ref[...])
    t = (acc_ref[0] + acc_ref[1] + g_ref[...] * dinv) * dinv
    out1 = jnp.maximum(t + b_ref[...], 0.0)
    h2 = jnp.dot(out1, w_ref[...], preferred_element_type=jnp.float32)
    g2_ref[...] = jnp.concatenate(
        [h2, jnp.zeros((NP, D1 - D2), jnp.float32)], axis=1)


def _tc_pool_classify(acc_ref, g_ref, dp0_ref, dp1_ref, b_ref, bat_ref,
                      seq_ref, wc_ref, bc_ref, o_ref):
    dinv = _dinv(dp0_ref[...], dp1_ref[...])
    t = (acc_ref[0] + acc_ref[1] + g_ref[...] * dinv) * dinv
    out2 = jnp.maximum(t + b_ref[...], 0.0)
    seg = bat_ref[...] * T + seq_ref[...]
    seg_ids = lax.broadcasted_iota(jnp.int32, (B * T, NP), 0)
    maskT = jnp.where(seg == seg_ids, 1.0, 0.0)
    pooled = jnp.dot(maskT, out2, preferred_element_type=jnp.float32,
                     precision=lax.Precision.HIGHEST)
    cnt = jnp.sum(maskT, axis=1, keepdims=True)
    mean = pooled / jnp.maximum(cnt, 1.0)
    ib = lax.broadcasted_iota(jnp.int32, (B, B * T), 0)
    ic = lax.broadcasted_iota(jnp.int32, (B, B * T), 1)
    pmat = jnp.where(ic // T == ib, 1.0 / T, 0.0)
    temp = jnp.dot(pmat, mean, preferred_element_type=jnp.float32,
                   precision=lax.Precision.HIGHEST)
    o_ref[...] = (jnp.dot(temp, wc_ref[...], preferred_element_type=jnp.float32,
                          precision=lax.Precision.HIGHEST) + bc_ref[...])


def kernel(x, edge_index, edge_attr, batch, seq, W1, b1, W2, b2, Wc, bc):
    pad = NP - N
    batch_p = jnp.pad(batch, (0, pad), constant_values=B)[None, :]
    seq_p = jnp.pad(seq, (0, pad))[None, :]
    b2_p = jnp.pad(b2, (0, D1 - D2))
    wc_p = jnp.pad(Wc, ((0, D1 - D2), (0, 0)))

    deg_p, meta = _sc_degree(edge_index[0], edge_index[1],
                             edge_attr.reshape(-1))
    h1 = pl.pallas_call(
        _tc_matmul_pad,
        out_shape=jax.ShapeDtypeStruct((NP, D1), jnp.float32),
    )(x, W1)
    dp0 = deg_p[0][:, None]
    dp1 = deg_p[1][:, None]

    deg_p3 = deg_p.reshape(NC, NP // 128, 128)

    acc1 = _sc_conv(h1, meta, deg_p3)

    g2 = pl.pallas_call(
        _tc_combine_matmul,
        out_shape=jax.ShapeDtypeStruct((NP, D1), jnp.float32),
    )(acc1, h1, dp0, dp1, b1, W2)

    acc2 = _sc_conv(g2, meta, deg_p3)

    out = pl.pallas_call(
        _tc_pool_classify,
        out_shape=jax.ShapeDtypeStruct((B, NUM_CLASSES), jnp.float32),
    )(acc2, g2, dp0, dp1, b2_p, batch_p, seq_p, wc_p, bc)

    return out


# speedup vs baseline: 1.0144x; 1.0144x over previous
import functools

import jax
import jax.numpy as jnp
from jax import lax
from jax.experimental import pallas as pl
from jax.experimental.pallas import tpu as pltpu
from jax.experimental.pallas import tpu_sc as plsc

N = 10000
NP = 10240
E = 320000
EP = 327680
D_FEAT = 128
D1 = 128
D2 = 64
B = 8
T = 16
NUM_CLASSES = 2

NC = 2
NS = 16
LANES = 16
NW = NC * NS
CH = 128
TRIPS = EP // (NW * CH)
RPT = NP // NS

_MESH = plsc.VectorSubcoreMesh(core_axis_name="c", subcore_axis_name="s")
_SC_PARAMS = pltpu.CompilerParams(needs_layout_passes=False)


NCH_REAL = E // CH
NCH_TOT = EP // CH
MROW = 3 * CH


@functools.partial(
    pl.kernel,
    out_type=(
        jax.ShapeDtypeStruct((NC, NP), jnp.float32),
        jax.ShapeDtypeStruct((NCH_TOT * MROW,), jnp.int32),
    ),
    mesh=_MESH,
    scratch_types=(
        pltpu.VMEM((2, CH), jnp.int32),
        pltpu.VMEM((2, CH), jnp.int32),
        pltpu.VMEM((2, CH * 4), jnp.float32),
        pltpu.VMEM((2, MROW), jnp.int32),
        pltpu.VMEM((2, CH), jnp.int32),
        pltpu.VMEM((2, CH), jnp.float32),
        pltpu.VMEM((RPT,), jnp.float32),
        pltpu.VMEM_SHARED((NP,), jnp.float32),
        pltpu.SemaphoreType.DMA,
        pltpu.SemaphoreType.DMA,
        pltpu.SemaphoreType.DMA,
        pltpu.SemaphoreType.DMA,
        pltpu.SemaphoreType.DMA,
        pltpu.SemaphoreType.DMA,
    ),
    compiler_params=_SC_PARAMS,
)
def _sc_degree(src_hbm, dst_hbm, ea_hbm, deg_out, meta_out,
               src_v, dst_v, ea_v, stage_v, dsti_v, ewf_v, zv, deg_sh,
               semi0, semi1, semw0, semw1, semc0, semc1):
    c = lax.axis_index("c")
    s = lax.axis_index("s")
    w = c * NS + s
    semi = (semi0, semi1)
    semw = (semw0, semw1)
    semc = (semc0, semc1)
    rt = 78 + jnp.where(w < NCH_REAL - 78 * NW, 1, 0)

    z16 = jnp.zeros((LANES,), jnp.float32)

    def zbody(i, carry):
        zv[pl.ds(i * LANES, LANES)] = z16
        return carry

    lax.fori_loop(0, RPT // LANES, zbody, 0)
    pltpu.sync_copy(zv, deg_sh.at[pl.ds(s * RPT, RPT)])
    plsc.subcore_barrier()

    lane_iota = lax.iota(jnp.int32, LANES)

    def issue_in(k, b):
        base = (w + k * NW) * CH
        pltpu.async_copy(src_hbm.at[pl.ds(base, CH)], src_v.at[b], semi[b])
        pltpu.async_copy(dst_hbm.at[pl.ds(base, CH)], dst_v.at[b], semi[b])
        pltpu.async_copy(ea_hbm.at[pl.ds(base * 4, CH * 4)], ea_v.at[b],
                         semi[b])

    def wait_in(b):
        pltpu.make_async_copy(src_hbm.at[pl.ds(0, CH)], src_v.at[b],
                              semi[b]).wait()
        pltpu.make_async_copy(dst_hbm.at[pl.ds(0, CH)], dst_v.at[b],
                              semi[b]).wait()
        pltpu.make_async_copy(ea_hbm.at[pl.ds(0, CH * 4)], ea_v.at[b],
                              semi[b]).wait()

    def meta_slot(k):
        return (w + k * NW) * MROW

    def wait_wr(b):
        pltpu.make_async_copy(stage_v.at[b], meta_out.at[pl.ds(0, MROW)],
                              semw[b]).wait()

    def wait_sc(b):
        pltpu.make_async_copy(ewf_v.at[b], deg_sh.at[dsti_v.at[b]],
                              semc[b]).wait()

    def body(k, b):
        in_range = k < rt

        @pl.when(in_range)
        def _():
            wait_in(b)

        @pl.when(jnp.logical_and(k < rt, k >= 2))
        def _():
            wait_wr(b)
            wait_sc(b)

        @pl.when(in_range)
        def _():
            for g in range(CH // LANES):
                sl = pl.ds(g * LANES, LANES)
                idx = lane_iota * 4 + (g * LANES * 4)
                ew16 = jnp.abs(plsc.load_gather(
                    ea_v, [jnp.full((LANES,), b, jnp.int32), idx]))
                stage_v[b, pl.ds(2 * CH + g * LANES, LANES)] = plsc.bitcast(
                    ew16, jnp.int32)
                ewf_v[b, sl] = ew16
                stage_v[b, sl] = src_v[b, sl]
                stage_v[b, pl.ds(CH + g * LANES, LANES)] = dst_v[b, sl]
                dsti_v[b, sl] = dst_v[b, sl]

        @pl.when(k + 2 < rt)
        def _():
            issue_in(k + 2, b)

        @pl.when(in_range)
        def _():
            pltpu.async_copy(stage_v.at[b],
                             meta_out.at[pl.ds(meta_slot(k), MROW)], semw[b])
            pltpu.async_copy(ewf_v.at[b], deg_sh.at[dsti_v.at[b]], semc[b],
                             add=True)

    issue_in(0, 0)
    issue_in(1, 1)

    def pair(j, carry):
        body(2 * j, 0)
        body(2 * j + 1, 1)
        return carry

    lax.fori_loop(0, (TRIPS + 1) // 2, pair, 0)
    wait_wr(0)
    wait_wr(1)
    wait_sc(0)
    wait_sc(1)

    def pad_chunk(p):
        for g in range(CH // LANES):
            sl = pl.ds(g * LANES, LANES)
            idx = N + ((lane_iota + g * LANES + p * CH) % (NP - N))
            stage_v[0, sl] = idx
            stage_v[0, pl.ds(CH + g * LANES, LANES)] = idx
            stage_v[0, pl.ds(2 * CH + g * LANES, LANES)] = jnp.zeros(
                (LANES,), jnp.int32)
        pltpu.sync_copy(stage_v.at[0], meta_out.at[pl.ds(p * MROW, MROW)])

    pad_chunk(NCH_REAL + w)

    @pl.when(w < NCH_TOT - NCH_REAL - NW)
    def _():
        pad_chunk(NCH_REAL + NW + w)

    plsc.subcore_barrier()
    pltpu.sync_copy(deg_sh.at[pl.ds(s * RPT, RPT)],
                    deg_out.at[c, pl.ds(s * RPT, RPT)])


@functools.partial(
    pl.kernel,
    out_type=jax.ShapeDtypeStruct((NC, NP, D1), jnp.float32),
    mesh=_MESH,
    scratch_types=(
        pltpu.VMEM((2, MROW), jnp.int32),
        pltpu.VMEM((2, CH), jnp.int32),
        pltpu.VMEM((2, CH), jnp.float32),
        pltpu.VMEM((2, CH, D1), jnp.float32),
        pltpu.VMEM((32, D1), jnp.float32),
        pltpu.VMEM_SHARED((NP, D1), jnp.float32),
        pltpu.SemaphoreType.DMA,
        pltpu.SemaphoreType.DMA,
        pltpu.SemaphoreType.DMA,
        pltpu.SemaphoreType.DMA,
        pltpu.SemaphoreType.DMA,
        pltpu.SemaphoreType.DMA,
        pltpu.SemaphoreType.DMA,
    ),
    compiler_params=_SC_PARAMS,
)
def _sc_conv(g_hbm, meta_hbm, out_hbm,
             meta_v, dsts_v, ews_v, rows_v, zrow_v,
             acc_sh, semm0, semm1, semg0, semg1, semt0, semt1, semz):
    c = lax.axis_index("c")
    s = lax.axis_index("s")
    w = c * NS + s
    semm = (semm0, semm1)
    semg = (semg0, semg1)
    semt = (semt0, semt1)

    z16 = jnp.zeros((LANES,), jnp.float32)

    def zbody(i, carry):
        for f in range(D1 // LANES):
            zrow_v[i, pl.ds(f * LANES, LANES)] = z16
        return carry

    lax.fori_loop(0, 32, zbody, 0)
    for i in range(RPT // 32):
        pltpu.async_copy(zrow_v, acc_sh.at[pl.ds(s * RPT + i * 32, 32)], semz)

    for i in range(RPT // 32):
        pltpu.make_async_copy(
            zrow_v, acc_sh.at[pl.ds(s * RPT + i * 32, 32)], semz).wait()
    plsc.subcore_barrier()

    def issue_meta(k, b):
        base = (w + k * NW) * MROW
        pltpu.async_copy(meta_hbm.at[pl.ds(base, MROW)], meta_v.at[b],
                         semm[b])

    def wait_meta(b):
        pltpu.make_async_copy(meta_hbm.at[pl.ds(0, MROW)], meta_v.at[b],
                              semm[b]).wait()

    def issue_gather(b):
        pltpu.async_copy(g_hbm.at[meta_v.at[b, pl.ds(0, CH)]], rows_v.at[b],
                         semg[b])

    def wait_gather(b):
        pltpu.make_async_copy(g_hbm.at[meta_v.at[b, pl.ds(0, CH)]],
                              rows_v.at[b], semg[b]).wait()

    def issue_scatter(b):
        pltpu.async_copy(rows_v.at[b], acc_sh.at[dsts_v.at[b]], semt[b],
                         add=True)

    def wait_scatter(b):
        pltpu.make_async_copy(rows_v.at[b], acc_sh.at[dsts_v.at[b]],
                              semt[b]).wait()

    def scale(b):
        def grp(gq, carry2):
            w16 = ews_v[b, pl.ds(gq * LANES, LANES)]
            for e in range(LANES):
                bc = jnp.take_along_axis(
                    w16, jnp.full((LANES,), e, jnp.int32), axis=0)
                r = gq * LANES + e
                for f in range(D1 // LANES):
                    sl = pl.ds(f * LANES, LANES)
                    rows_v[b, r, sl] = rows_v[b, r, sl] * bc
            return carry2

        lax.fori_loop(0, CH // LANES, grp, 0)

    def body(k, b, first, last):
        nb = 1 - b
        wait_gather(b)
        for f in range(CH // LANES):
            sl = pl.ds(f * LANES, LANES)
            dsts_v[b, sl] = meta_v[b, pl.ds(CH + f * LANES, LANES)]
            ews_v[b, sl] = plsc.bitcast(
                meta_v[b, pl.ds(2 * CH + f * LANES, LANES)], jnp.float32)

        @pl.when(k + 1 < TRIPS)
        def _():
            wait_meta(nb)

        if first:
            pass
        else:
            wait_scatter(nb)

        @pl.when(k + 1 < TRIPS)
        def _():
            issue_gather(nb)

        @pl.when(k + 2 < TRIPS)
        def _():
            issue_meta(k + 2, b)

        scale(b)
        issue_scatter(b)

    issue_meta(0, 0)
    issue_meta(1, 1)
    wait_meta(0)
    issue_gather(0)

    def pair(j, carry):
        body(2 * j, 0, first=False, last=False)
        body(2 * j + 1, 1, first=False, last=False)
        return carry

    body(0, 0, first=True, last=False)
    body(1, 1, first=False, last=False)
    lax.fori_loop(1, TRIPS // 2, pair, 0)
    wait_scatter(1)
    plsc.subcore_barrier()
    pltpu.sync_copy(acc_sh.at[pl.ds(s * RPT, RPT)],
                    out_hbm.at[c, pl.ds(s * RPT, RPT)])


def _tc_prep(x_ref, w_ref, dp0_ref, dp1_ref, dinv_ref, g_ref):
    h1 = jnp.dot(x_ref[...], w_ref[...], preferred_element_type=jnp.float32)
    dinv = lax.rsqrt(dp0_ref[...] + dp1_ref[...] + 1.0)
    dinv_ref[...] = dinv
    g_ref[pl.ds(0, N), :] = h1 * dinv[:N]
    g_ref[pl.ds(N, NP - N), :] = jnp.zeros((NP - N, D1), jnp.float32)


def _tc_combine_matmul(acc_ref, g_ref, dinv_ref, b_ref, w_ref, g2_ref):
    t = (acc_ref[0] + acc_ref[1] + g_ref[...]) * dinv_ref[...]
    out1 = jnp.maximum(t + b_ref[...], 0.0)
    h2 = jnp.dot(out1, w_ref[...], preferred_element_type=jnp.float32)
    g2_ref[...] = jnp.concatenate(
        [h2 * dinv_ref[...], jnp.zeros((NP, D1 - D2), jnp.float32)], axis=1)


def _tc_pool_classify(acc_ref, g_ref, dinv_ref, b_ref, bat_ref,
                      seq_ref, wc_ref, bc_ref, o_ref):
    t = (acc_ref[0] + acc_ref[1] + g_ref[...]) * dinv_ref[...]
    out2 = jnp.maximum(t + b_ref[...], 0.0)
    seg = bat_ref[...] * T + seq_ref[...]
    seg_ids = lax.broadcasted_iota(jnp.int32, (B * T, NP), 0)
    maskT = jnp.where(seg == seg_ids, 1.0, 0.0)
    pooled = jnp.dot(maskT, out2, preferred_element_type=jnp.float32,
                     precision=lax.Precision.HIGHEST)
    cnt = jnp.sum(maskT, axis=1, keepdims=True)
    mean = pooled / jnp.maximum(cnt, 1.0)
    ib = lax.broadcasted_iota(jnp.int32, (B, B * T), 0)
    ic = lax.broadcasted_iota(jnp.int32, (B, B * T), 1)
    pmat = jnp.where(ic // T == ib, 1.0 / T, 0.0)
    temp = jnp.dot(pmat, mean, preferred_element_type=jnp.float32,
                   precision=lax.Precision.HIGHEST)
    o_ref[...] = (jnp.dot(temp, wc_ref[...], preferred_element_type=jnp.float32,
                          precision=lax.Precision.HIGHEST) + bc_ref[...])


def kernel(x, edge_index, edge_attr, batch, seq, W1, b1, W2, b2, Wc, bc):
    pad = NP - N
    batch_p = jnp.pad(batch, (0, pad), constant_values=B)[None, :]
    seq_p = jnp.pad(seq, (0, pad))[None, :]
    b2_p = jnp.pad(b2, (0, D1 - D2))
    wc_p = jnp.pad(Wc, ((0, D1 - D2), (0, 0)))

    deg_p, meta = _sc_degree(edge_index[0], edge_index[1],
                             edge_attr.reshape(-1))

    dinv, g1 = pl.pallas_call(
        _tc_prep,
        out_shape=(
            jax.ShapeDtypeStruct((NP, 1), jnp.float32),
            jax.ShapeDtypeStruct((NP, D1), jnp.float32),
        ),
    )(x, W1, deg_p[0][:, None], deg_p[1][:, None])

    acc1 = _sc_conv(g1, meta)

    g2 = pl.pallas_call(
        _tc_combine_matmul,
        out_shape=jax.ShapeDtypeStruct((NP, D1), jnp.float32),
    )(acc1, g1, dinv, b1, W2)

    acc2 = _sc_conv(g2, meta)

    out = pl.pallas_call(
        _tc_pool_classify,
        out_shape=jax.ShapeDtypeStruct((B, NUM_CLASSES), jnp.float32),
    )(acc2, g2, dinv, b2_p, batch_p, seq_p, wc_p, bc)

    return out
